# baseline (speedup 1.0000x reference)
import functools

import jax
import jax.numpy as jnp
from jax import lax
from jax.experimental import pallas as pl
from jax.experimental.pallas import tpu as pltpu
from jax.experimental.pallas import tpu_sc as plsc

NUM_EXPERTS = 64
HIDDEN = 4096
TOKENS = 8192
SINKHORN_ITERS = 30
TILE = 512
NW = 16
TPW = TOKENS // NW
NCH = TPW // 16
NEC = NUM_EXPERTS // 16


def _router_kernel(x0_ref, x1_ref, w_ref, b_ref, logits_ref, aff_ref,
                   cost3_ref):
    w = w_ref[...]
    b = b_ref[...]
    half = TILE // 2
    for j, x_ref in enumerate((x0_ref, x1_ref)):
        x = x_ref[...].reshape(half, HIDDEN)
        logits = jax.lax.dot_general(
            x, w, (((1,), (0,)), ((), ())),
            preferred_element_type=jnp.float32,
        ) + b
        logits_ref[pl.ds(j * half, half), :] = logits
        aff_ref[pl.ds(j * half, half), :] = jax.nn.sigmoid(logits)
        cost3_ref[0, :, pl.ds(j * half, half)] = jnp.exp(logits).T


def _sc_sinkhorn(cost3_hbm, idx_hbm, stage_hbm, C_v, d0_v, d1s_v, part_v,
                 grid_v, idx_v):
    wid = lax.axis_index("s")
    pltpu.sync_copy(cost3_hbm.at[wid], C_v)
    eps = 1e-8
    lane = lax.broadcasted_iota(jnp.int32, (16,), 0)
    ones = jnp.ones((16,), jnp.float32)
    for ec in range(NEC):
        d1s_v[pl.ds(ec * 16, 16)] = ones

    def iteration(_, tok):
        def p1_body(ec, accs):
            d1c = d1s_v[pl.ds(ec * 16, 16)]
            for k in range(16):
                e = ec * 16 + k
                d1e = d1c[k]
                accs = tuple(accs[c] + d1e * C_v[e, pl.ds(c * 16, 16)]
                             for c in range(NCH))
            return accs

        accs = lax.fori_loop(
            0, NEC, p1_body,
            tuple(jnp.zeros((16,), jnp.float32) for _ in range(NCH)))
        for c in range(NCH):
            d0_v[pl.ds(c * 16, 16)] = (1.0 / TOKENS) / (accs[c] + eps)

        def p2_body(ec, tok):
            part = jnp.zeros((16,), jnp.float32)
            for k in range(16):
                e = ec * 16 + k
                acc = jnp.zeros((16,), jnp.float32)
                for c in range(NCH):
                    acc = acc + (C_v[e, pl.ds(c * 16, 16)]
                                 * d0_v[pl.ds(c * 16, 16)])
                part = part + jnp.where(lane == k, jnp.sum(acc), 0.0)
            part_v[pl.ds(ec * 16, 16)] = part
            return tok

        tok = lax.fori_loop(0, NEC, p2_body, tok)

        pltpu.sync_copy(part_v, stage_hbm.at[wid])
        plsc.subcore_barrier()
        pltpu.sync_copy(stage_hbm, grid_v)

        def rloop(ec, tok):
            def rbody(w, v):
                return v + grid_v[w, pl.ds(ec * 16, 16)]
            v = lax.fori_loop(0, NW, rbody, jnp.zeros((16,), jnp.float32))
            d1s_v[pl.ds(ec * 16, 16)] = (1.0 / NUM_EXPERTS) / (v + eps)
            return tok

        tok = lax.fori_loop(0, NEC, rloop, tok)
        plsc.subcore_barrier()
        return tok

    lax.fori_loop(0, SINKHORN_ITERS, iteration, 0)

    def am_body(ec, carry):
        best, bidx = carry
        d1c = d1s_v[pl.ds(ec * 16, 16)]
        for k in range(16):
            e = ec * 16 + k
            d1e = d1c[k]
            nbest, nbidx = [], []
            for c in range(NCH):
                val = d1e * C_v[e, pl.ds(c * 16, 16)]
                take = val > best[c]
                nbest.append(jnp.where(take, val, best[c]))
                nbidx.append(jnp.where(take,
                                       jnp.zeros((16,), jnp.int32) + e,
                                       bidx[c]))
            best, bidx = tuple(nbest), tuple(nbidx)
        return best, bidx

    best0 = tuple(jnp.full((16,), -jnp.inf, jnp.float32) for _ in range(NCH))
    bidx0 = tuple(jnp.zeros((16,), jnp.int32) for _ in range(NCH))
    _, bidx = lax.fori_loop(0, NEC, am_body, (best0, bidx0))
    for c in range(NCH):
        idx_v[pl.ds(c * 16, 16)] = bidx[c]
    pltpu.sync_copy(idx_v, idx_hbm.at[wid])


@functools.partial(jax.jit, static_argnames=())
def kernel(hidden_states, W, b):
    n_tiles = TOKENS // TILE
    logits, aff, cost3 = pl.pallas_call(
        _router_kernel,
        grid=(n_tiles,),
        in_specs=[
            pl.BlockSpec((TILE // 8, 4, HIDDEN), lambda i: (2 * i, 0, 0)),
            pl.BlockSpec((TILE // 8, 4, HIDDEN), lambda i: (2 * i + 1, 0, 0)),
            pl.BlockSpec((HIDDEN, NUM_EXPERTS), lambda i: (0, 0)),
            pl.BlockSpec((1, NUM_EXPERTS), lambda i: (0, 0)),
        ],
        out_specs=[
            pl.BlockSpec((TILE, NUM_EXPERTS), lambda i: (i, 0)),
            pl.BlockSpec((TILE, NUM_EXPERTS), lambda i: (i, 0)),
            pl.BlockSpec((1, NUM_EXPERTS, TPW), lambda i: (i, 0, 0)),
        ],
        out_shape=[
            jax.ShapeDtypeStruct((TOKENS, NUM_EXPERTS), jnp.float32),
            jax.ShapeDtypeStruct((TOKENS, NUM_EXPERTS), jnp.float32),
            jax.ShapeDtypeStruct((NW, NUM_EXPERTS, TPW), jnp.float32),
        ],
    )(hidden_states, hidden_states, W, b.reshape(1, NUM_EXPERTS))

    sc = functools.partial(
        pl.kernel,
        mesh=plsc.VectorSubcoreMesh(core_axis_name="c", subcore_axis_name="s",
                                    num_cores=1),
        compiler_params=pltpu.CompilerParams(needs_layout_passes=False),
        out_type=[jax.ShapeDtypeStruct((NW, TPW), jnp.int32),
                  jax.ShapeDtypeStruct((NW, NUM_EXPERTS), jnp.float32)],
        scratch_types=[
            pltpu.VMEM((NUM_EXPERTS, TPW), jnp.float32),
            pltpu.VMEM((TPW,), jnp.float32),
            pltpu.VMEM((NUM_EXPERTS,), jnp.float32),
            pltpu.VMEM((NUM_EXPERTS,), jnp.float32),
            pltpu.VMEM((NW, NUM_EXPERTS), jnp.float32),
            pltpu.VMEM((TPW,), jnp.int32),
        ],
    )(_sc_sinkhorn)
    idx, _ = sc(cost3)

    return (logits, aff, idx.reshape(TOKENS, 1))


# speedup vs baseline: 6.0332x; 6.0332x over previous
import functools

import jax
import jax.numpy as jnp
from jax.experimental import pallas as pl
from jax.experimental.pallas import tpu as pltpu

NUM_EXPERTS = 64
HIDDEN = 4096
TOKENS = 8192
SINKHORN_ITERS = 30
TILE = 512
CHUNK = 512


def _router_kernel(x0_ref, x1_ref, w_ref, b_ref, logits_ref, aff_ref, idx_ref,
                   costT_ref):
    i = pl.program_id(0)
    w = w_ref[...]
    b = b_ref[...]
    half = TILE // 2
    for j, x_ref in enumerate((x0_ref, x1_ref)):
        x = x_ref[...].reshape(half, HIDDEN)
        logits = jax.lax.dot_general(
            x, w, (((1,), (0,)), ((), ())),
            preferred_element_type=jnp.float32,
        ) + b
        logits_ref[pl.ds(j * half, half), :] = logits
        aff_ref[pl.ds(j * half, half), :] = jax.nn.sigmoid(logits)
        costT_ref[:, pl.ds(i * TILE + j * half, half)] = jnp.exp(logits).T

    @pl.when(i == pl.num_programs(0) - 1)
    def _sinkhorn():
        eps = 1e-8
        n_ch = TOKENS // CHUNK

        def body(_, d1):
            v = jnp.zeros((NUM_EXPERTS, 1), jnp.float32)
            for c in range(n_ch):
                blk = costT_ref[:, c * CHUNK:(c + 1) * CHUNK]
                s = jnp.sum(blk * d1, axis=0, keepdims=True)
                d0 = (1.0 / TOKENS) / (s + eps)
                v = v + jnp.sum(blk * d0, axis=1, keepdims=True)
            return (1.0 / NUM_EXPERTS) / (v + eps)

        d1 = jax.lax.fori_loop(0, SINKHORN_ITERS, body,
                               jnp.ones((NUM_EXPERTS, 1), jnp.float32))
        m = costT_ref[...] * d1
        maxv = jnp.max(m, axis=0, keepdims=True)
        eidx = jax.lax.broadcasted_iota(jnp.int32, (NUM_EXPERTS, TOKENS), 0)
        idx_ref[...] = jnp.min(
            jnp.where(m == maxv, eidx, NUM_EXPERTS), axis=0, keepdims=True)


@functools.partial(jax.jit, static_argnames=())
def kernel(hidden_states, W, b):
    n_tiles = TOKENS // TILE
    logits, aff, idx = pl.pallas_call(
        _router_kernel,
        grid=(n_tiles,),
        in_specs=[
            pl.BlockSpec((TILE // 8, 4, HIDDEN), lambda i: (2 * i, 0, 0)),
            pl.BlockSpec((TILE // 8, 4, HIDDEN), lambda i: (2 * i + 1, 0, 0)),
            pl.BlockSpec((HIDDEN, NUM_EXPERTS), lambda i: (0, 0)),
            pl.BlockSpec((1, NUM_EXPERTS), lambda i: (0, 0)),
        ],
        out_specs=[
            pl.BlockSpec((TILE, NUM_EXPERTS), lambda i: (i, 0)),
            pl.BlockSpec((TILE, NUM_EXPERTS), lambda i: (i, 0)),
            pl.BlockSpec((1, TOKENS), lambda i: (0, 0)),
        ],
        out_shape=[
            jax.ShapeDtypeStruct((TOKENS, NUM_EXPERTS), jnp.float32),
            jax.ShapeDtypeStruct((TOKENS, NUM_EXPERTS), jnp.float32),
            jax.ShapeDtypeStruct((1, TOKENS), jnp.int32),
        ],
        scratch_shapes=[pltpu.VMEM((NUM_EXPERTS, TOKENS), jnp.float32)],
    )(hidden_states, hidden_states, W, b.reshape(1, NUM_EXPERTS))

    return (logits, aff, idx.reshape(TOKENS, 1))
